# baseline (speedup 1.0000x reference)
import functools

import jax
import jax.numpy as jnp
from jax import lax
from jax.experimental import pallas as pl
from jax.experimental.pallas import tpu as pltpu
from jax.experimental.pallas import tpu_sc as plsc

N = 10000
E = 320000
D = 128
F32 = jnp.float32
BF16 = jnp.bfloat16
I32 = jnp.int32

NW = 32
EPW = E // NW
CH3 = 2000
CW = 80
NCH = EPW // CW
BN_E = 2000
BU_E = 2560
BN_N = 2000
RPT = 624
EPS = 1e-5

_sc_mesh = plsc.VectorSubcoreMesh(core_axis_name="c", subcore_axis_name="s")
_sc_params = pltpu.CompilerParams(needs_layout_passes=False)


def _k1_body(h_ref, wh_ref, wsd_ref, wpq_ref, we_ref, wpe_ref, wue_ref,
             zh_ref, as_ref, ad_ref, pq_ref, me_ref, mu_ref):
    zh = jnp.dot(h_ref[...], wh_ref[...], preferred_element_type=F32)
    zh_ref[...] = zh
    as_ref[...] = jnp.dot(zh, wsd_ref[:D], preferred_element_type=F32)
    ad_ref[...] = jnp.dot(zh, wsd_ref[D:], preferred_element_type=F32)
    pq_ref[...] = lax.dot_general(wpq_ref[...], zh, (((0,), (1,)), ((), ())),
                                  preferred_element_type=F32)
    me_ref[...] = jnp.dot(we_ref[...], wpe_ref[...], preferred_element_type=F32)
    mu_ref[...] = jnp.dot(we_ref[...], wue_ref[...], preferred_element_type=F32)


_k1 = pl.pallas_call(
    _k1_body,
    out_shape=[
        jax.ShapeDtypeStruct((N, D), F32),
        jax.ShapeDtypeStruct((N, D), F32),
        jax.ShapeDtypeStruct((N, D), F32),
        jax.ShapeDtypeStruct((8, N), F32),
        jax.ShapeDtypeStruct((D, D), F32),
        jax.ShapeDtypeStruct((D, 8), F32),
    ],
)


def _k2_body(e_ref, me_ref, mu_ref, c_ref, u_ref):
    x = e_ref[...]
    c_ref[...] = jnp.dot(x, me_ref[...], preferred_element_type=F32)
    u_ref[...] = lax.dot_general(mu_ref[...], x, (((0,), (1,)), ((), ())),
                                 preferred_element_type=F32)


_k2 = pl.pallas_call(
    _k2_body,
    grid=(E // BU_E,),
    in_specs=[
        pl.BlockSpec((BU_E, D), lambda i: (i, 0)),
        pl.BlockSpec((D, D), lambda i: (0, 0)),
        pl.BlockSpec((D, 8), lambda i: (0, 0)),
    ],
    out_specs=[
        pl.BlockSpec((BU_E, D), lambda i: (i, 0)),
        pl.BlockSpec((8, BU_E), lambda i: (0, i)),
    ],
    out_shape=[
        jax.ShapeDtypeStruct((E, D), F32),
        jax.ShapeDtypeStruct((8, E), F32),
    ],
)


@functools.partial(
    pl.kernel,
    mesh=_sc_mesh,
    compiler_params=_sc_params,
    out_type=[
        jax.ShapeDtypeStruct((E,), F32),
        jax.ShapeDtypeStruct((NW * N,), F32),
    ],
    scratch_types=[
        pltpu.VMEM((N,), F32),
        pltpu.VMEM((N,), F32),
        pltpu.VMEM((N,), F32),
        pltpu.VMEM((CH3,), F32),
        pltpu.VMEM((CH3,), I32),
        pltpu.VMEM((CH3,), I32),
        pltpu.VMEM((CH3,), F32),
    ],
)
def _k3(u_hbm, pq_hbm, ei_hbm, ex_hbm, den_hbm,
        p_v, q_v, den_v, u_v, s_v, d_v, ex_v):
    wid = lax.axis_index("s") * 2 + lax.axis_index("c")
    pltpu.sync_copy(pq_hbm.at[pl.ds(0, N)], p_v)
    pltpu.sync_copy(pq_hbm.at[pl.ds(N, N)], q_v)

    def _zero(i, carry):
        den_v[pl.ds(i * 16, 16)] = jnp.zeros((16,), F32)
        return carry

    lax.fori_loop(0, N // 16, _zero, 0)

    def _chunk(c, carry):
        base = wid * EPW + c * CH3
        pltpu.sync_copy(u_hbm.at[pl.ds(base, CH3)], u_v)
        pltpu.sync_copy(ei_hbm.at[pl.ds(base, CH3)], s_v)
        pltpu.sync_copy(ei_hbm.at[pl.ds(E + base, CH3)], d_v)

        def _inner(j, icarry):
            off = j * 16
            s = s_v[pl.ds(off, 16)]
            dd = d_v[pl.ds(off, 16)]
            pg = plsc.load_gather(p_v, [s])
            qg = plsc.load_gather(q_v, [dd])
            a = jnp.maximum(u_v[pl.ds(off, 16)] + pg + qg, 0.0)
            ev = jnp.exp(a)
            ex_v[pl.ds(off, 16)] = ev
            plsc.addupdate_scatter(den_v, [dd], ev)
            return icarry

        lax.fori_loop(0, CH3 // 16, _inner, 0)
        pltpu.sync_copy(ex_v, ex_hbm.at[pl.ds(base, CH3)])
        return carry

    lax.fori_loop(0, EPW // CH3, _chunk, 0)
    pltpu.sync_copy(den_v, den_hbm.at[pl.ds(wid * N, N)])


@functools.partial(
    pl.kernel,
    mesh=_sc_mesh,
    compiler_params=_sc_params,
    out_type=jax.ShapeDtypeStruct((2, N, D), F32),
    scratch_types=[
        pltpu.VMEM((EPW,), I32),
        pltpu.VMEM((EPW,), I32),
        pltpu.VMEM((EPW,), F32),
        pltpu.VMEM((CW, D), F32),
        pltpu.VMEM((CW, D), F32),
        pltpu.VMEM_SHARED((N, D), F32),
        pltpu.SemaphoreType.DMA,
        pltpu.SemaphoreType.DMA,
    ],
)
def _k4(ex_hbm, ei_hbm, zh_hbm, out_hbm,
        s1d, d1d, e1d, rows0, rows1, hsh, semf0, semf1):
    cid = lax.axis_index("c")
    sid = lax.axis_index("s")
    wid = sid * 2 + cid

    for i in range(16):
        for g in range(D // 16):
            rows1[i, pl.ds(g * 16, 16)] = jnp.zeros((16,), F32)
    tstart = sid * RPT
    nz = (RPT // 16) + jnp.where(sid == 15, 1, 0)

    def _zfill(t, carry):
        pltpu.sync_copy(rows1.at[pl.ds(0, 16)], hsh.at[pl.ds(tstart + t * 16, 16)])
        return carry

    lax.fori_loop(0, nz, _zfill, 0)
    plsc.subcore_barrier()

    ebase = wid * EPW
    pltpu.sync_copy(ei_hbm.at[pl.ds(ebase, EPW)], s1d)
    pltpu.sync_copy(ei_hbm.at[pl.ds(E + ebase, EPW)], d1d)
    pltpu.sync_copy(ex_hbm.at[pl.ds(ebase, EPW)], e1d)

    bufs = (rows0, rows1)
    sems = (semf0, semf1)

    def _fetch(c, buf, sem):
        pltpu.async_copy(zh_hbm.at[s1d.at[pl.ds(c * CW, CW)]], buf, sem)

    def _waitf(buf, sem):
        pltpu.make_async_copy(zh_hbm.at[s1d.at[pl.ds(0, CW)]], buf, sem).wait()

    def _process(c, buf):
        jb = c * CW

        def _row(r, rcarry):
            a = plsc.load_gather(e1d, [jnp.full((16,), jb + r, I32)])
            for g in range(D // 16):
                sl = pl.ds(g * 16, 16)
                buf[r, sl] = buf[r, sl] * a
            return rcarry

        lax.fori_loop(0, CW, _row, 0, unroll=2)
        pltpu.sync_copy(buf, hsh.at[d1d.at[pl.ds(jb, CW)]], add=True)

    _fetch(0, rows0, semf0)
    _fetch(1, rows1, semf1)

    def _pair(k, carry):
        for b in range(2):
            c = k * 2 + b
            _waitf(bufs[b], sems[b])
            _process(c, bufs[b])
            if b == 0:
                _fetch(c + 2, bufs[b], sems[b])
            else:
                @pl.when(k != (NCH // 2) - 1)
                def _():
                    _fetch(c + 2, bufs[b], sems[b])
        return carry

    lax.fori_loop(0, NCH // 2, _pair, 0)
    _waitf(rows0, semf0)
    _process(NCH - 1, rows0)

    plsc.subcore_barrier()

    def _cpout(t, carry):
        roff = tstart + t * 16
        pltpu.sync_copy(hsh.at[pl.ds(roff, 16)], out_hbm.at[cid, pl.ds(roff, 16)])
        return carry

    lax.fori_loop(0, nz, _cpout, 0)


@functools.partial(
    pl.kernel,
    mesh=_sc_mesh,
    compiler_params=_sc_params,
    out_type=[
        jax.ShapeDtypeStruct((E, D), F32),
        jax.ShapeDtypeStruct((NW, 2, D), F32),
    ],
    scratch_types=[
        pltpu.VMEM((EPW,), I32),
        pltpu.VMEM((EPW,), I32),
        pltpu.VMEM((CW, D), F32),
        pltpu.VMEM((CW, D), F32),
        pltpu.VMEM((CW, D), F32),
        pltpu.VMEM((CW, D), F32),
        pltpu.VMEM((CW, D), F32),
        pltpu.VMEM((CW, D), F32),
        pltpu.VMEM((CW, D), F32),
        pltpu.VMEM((CW, D), F32),
        pltpu.VMEM((D,), F32),
        pltpu.VMEM((D,), F32),
        pltpu.SemaphoreType.DMA,
        pltpu.SemaphoreType.DMA,
        pltpu.SemaphoreType.DMA,
        pltpu.SemaphoreType.DMA,
    ],
)
def _k6a(c_hbm, as_hbm, ad_hbm, ei_hbm, ep_hbm, sums_hbm,
         s1d, d1d, cbuf0, cbuf1, rs0, rs1, rd0, rd1, obuf0, obuf1,
         st1, st2, semf0, semf1, sems0, sems1):
    cid = lax.axis_index("c")
    sid = lax.axis_index("s")
    wid = sid * 2 + cid

    ebase = wid * EPW
    pltpu.sync_copy(ei_hbm.at[pl.ds(ebase, EPW)], s1d)
    pltpu.sync_copy(ei_hbm.at[pl.ds(E + ebase, EPW)], d1d)

    ins = ((cbuf0, rs0, rd0), (cbuf1, rs1, rd1))
    outs = (obuf0, obuf1)
    fsems = (semf0, semf1)
    ssems = (sems0, sems1)
    zero16 = jnp.zeros((16,), F32)

    def _fetch(c, b):
        jb = c * CW
        cb, rsb, rdb = ins[b]
        pltpu.async_copy(c_hbm.at[pl.ds(ebase + jb, CW)], cb, fsems[b])
        pltpu.async_copy(as_hbm.at[s1d.at[pl.ds(jb, CW)]], rsb, fsems[b])
        pltpu.async_copy(ad_hbm.at[d1d.at[pl.ds(jb, CW)]], rdb, fsems[b])

    def _waitf(b):
        cb, rsb, rdb = ins[b]
        pltpu.make_async_copy(c_hbm.at[pl.ds(0, CW)], cb, fsems[b]).wait()
        pltpu.make_async_copy(as_hbm.at[pl.ds(0, CW)], rsb, fsems[b]).wait()
        pltpu.make_async_copy(ad_hbm.at[pl.ds(0, CW)], rdb, fsems[b]).wait()

    def _waits(b):
        pltpu.make_async_copy(outs[b], ep_hbm.at[pl.ds(0, CW)], ssems[b]).wait()

    def _compute(b, carry):
        cb, rsb, rdb = ins[b]
        ob = outs[b]

        def _row(r, rcarry):
            nxt = []
            for g in range(D // 16):
                sl = pl.ds(g * 16, 16)
                v = cb[r, sl] + rsb[r, sl] + rdb[r, sl]
                ob[r, sl] = v
                nxt.append(rcarry[g] + v)
                nxt.append(rcarry[8 + g] + v * v)
            return tuple(nxt[0::2]) + tuple(nxt[1::2])

        return lax.fori_loop(0, CW, _row, carry, unroll=2)

    _fetch(0, 0)
    _fetch(1, 1)
    init = tuple(zero16 for _ in range(16))

    def _pair(k, carry):
        for b in range(2):
            c = k * 2 + b
            _waitf(b)

            @pl.when(k != 0)
            def _():
                _waits(b)

            carry = _compute(b, carry)
            if b == 0:
                _fetch(c + 2, b)
            else:
                @pl.when(k != (NCH // 2) - 1)
                def _():
                    _fetch(c + 2, b)
            pltpu.async_copy(outs[b], ep_hbm.at[pl.ds(ebase + c * CW, CW)],
                             ssems[b])
        return carry

    carry = lax.fori_loop(0, NCH // 2, _pair, init)
    _waitf(0)
    _waits(0)
    carry = _compute(0, carry)
    pltpu.async_copy(outs[0], ep_hbm.at[pl.ds(ebase + (NCH - 1) * CW, CW)],
                     sems0)
    for g in range(D // 16):
        st1[pl.ds(g * 16, 16)] = carry[g]
        st2[pl.ds(g * 16, 16)] = carry[8 + g]
    _waits(1)
    _waits(0)
    pltpu.sync_copy(st1, sums_hbm.at[wid, 0])
    pltpu.sync_copy(st2, sums_hbm.at[wid, 1])


def _k5_body(hacc_ref, den_ref, g_ref, b_ref, out_ref):
    hacc = hacc_ref[0] + hacc_ref[1]
    den = jnp.sum(den_ref[...], axis=0)
    den = jnp.where(den == 0.0, 1.0, den)
    hn = hacc / den[:, None]
    mu = jnp.mean(hn, axis=0, keepdims=True)
    var = jnp.mean(hn * hn, axis=0, keepdims=True) - mu * mu
    scale = g_ref[...] * lax.rsqrt(var + EPS)
    out_ref[...] = jnp.maximum((hn - mu) * scale + b_ref[...], 0.0)


_k5 = pl.pallas_call(
    _k5_body,
    out_shape=jax.ShapeDtypeStruct((N, D), F32),
)


def _k6b_body(ep_ref, sums_ref, g_ref, b_ref, out_ref):
    s = sums_ref[...]
    s1 = jnp.sum(s[:, 0, :], axis=0, keepdims=True)
    s2 = jnp.sum(s[:, 1, :], axis=0, keepdims=True)
    mu = s1 * (1.0 / E)
    var = s2 * (1.0 / E) - mu * mu
    scale = g_ref[...] * lax.rsqrt(var + EPS)
    out_ref[...] = jnp.maximum((ep_ref[...] - mu) * scale + b_ref[...], 0.0)


_k6b = pl.pallas_call(
    _k6b_body,
    grid=(E // BN_E,),
    in_specs=[
        pl.BlockSpec((BN_E, D), lambda i: (i, 0)),
        pl.BlockSpec((NW, 2, D), lambda i: (0, 0, 0)),
        pl.BlockSpec((1, D), lambda i: (0, 0)),
        pl.BlockSpec((1, D), lambda i: (0, 0)),
    ],
    out_specs=pl.BlockSpec((BN_E, D), lambda i: (i, 0)),
    out_shape=jax.ShapeDtypeStruct((E, D), F32),
)


def kernel(h, e, edge_index, W_h, W_e, W_p, b_p, W_a,
           gamma_h, beta_h, gamma_e, beta_e):
    del b_p
    ei = edge_index.reshape(2 * E)
    wpq = jnp.zeros((D, 8), F32)
    wpq = wpq.at[:, 0].set(W_a[D:2 * D, 0]).at[:, 1].set(W_a[2 * D:, 0])
    wue = jnp.zeros((D, 8), F32).at[:, 0].set(W_a[:D, 0])

    z_h, a_s, a_d, pqt, m_e, mu8 = _k1(h, W_h, W_p[D:], wpq, W_e, W_p[:D], wue)
    c, ut = _k2(e, m_e, mu8)

    ex, den_flat = _k3(ut.reshape(8 * E), pqt.reshape(8 * N), ei)

    eproj, sums = _k6a(c, a_s, a_d, ei)
    hacc2 = _k4(ex, ei, z_h)

    h_out = _k5(hacc2, den_flat.reshape(NW, N), gamma_h.reshape(1, D),
                beta_h.reshape(1, D))
    e_out = _k6b(eproj, sums, gamma_e.reshape(1, D), beta_e.reshape(1, D))
    return h_out, e_out


# speedup vs baseline: 1.3985x; 1.3985x over previous
import functools

import jax
import jax.numpy as jnp
from jax import lax
from jax.experimental import pallas as pl
from jax.experimental.pallas import tpu as pltpu
from jax.experimental.pallas import tpu_sc as plsc

N = 10000
E = 320000
D = 128
F32 = jnp.float32
BF16 = jnp.bfloat16
I32 = jnp.int32

NW = 32
EPW = E // NW
CH3 = 2000
CW = 80
NCH = EPW // CW
BN_E = 2000
BU_E = 2560
BN_N = 2000
RPT = 624
EPS = 1e-5

_sc_mesh = plsc.VectorSubcoreMesh(core_axis_name="c", subcore_axis_name="s")
_sc_params = pltpu.CompilerParams(needs_layout_passes=False)


def _k1_body(h_ref, wh_ref, wsd_ref, wpq_ref, we_ref, wpe_ref, wue_ref,
             zh_ref, as_ref, ad_ref, pq_ref, me_ref, mu_ref):
    zh = jnp.dot(h_ref[...], wh_ref[...], preferred_element_type=F32)
    zh_ref[...] = zh
    as_ref[...] = jnp.dot(zh, wsd_ref[:D], preferred_element_type=F32)
    ad_ref[...] = jnp.dot(zh, wsd_ref[D:], preferred_element_type=F32)
    pq_ref[...] = lax.dot_general(wpq_ref[...], zh, (((0,), (1,)), ((), ())),
                                  preferred_element_type=F32)
    me_ref[...] = jnp.dot(we_ref[...], wpe_ref[...], preferred_element_type=F32)
    mu_ref[...] = jnp.dot(we_ref[...], wue_ref[...], preferred_element_type=F32)


_k1 = pl.pallas_call(
    _k1_body,
    out_shape=[
        jax.ShapeDtypeStruct((N, D), F32),
        jax.ShapeDtypeStruct((N, D), F32),
        jax.ShapeDtypeStruct((N, D), F32),
        jax.ShapeDtypeStruct((8, N), F32),
        jax.ShapeDtypeStruct((D, D), F32),
        jax.ShapeDtypeStruct((D, 8), F32),
    ],
)


def _k2_body(e_ref, me_ref, mu_ref, c_ref, u_ref):
    x = e_ref[...]
    c_ref[...] = jnp.dot(x, me_ref[...], preferred_element_type=F32)
    u_ref[...] = lax.dot_general(mu_ref[...], x, (((0,), (1,)), ((), ())),
                                 preferred_element_type=F32)


_k2 = pl.pallas_call(
    _k2_body,
    grid=(E // BU_E,),
    in_specs=[
        pl.BlockSpec((BU_E, D), lambda i: (i, 0)),
        pl.BlockSpec((D, D), lambda i: (0, 0)),
        pl.BlockSpec((D, 8), lambda i: (0, 0)),
    ],
    out_specs=[
        pl.BlockSpec((BU_E, D), lambda i: (i, 0)),
        pl.BlockSpec((8, BU_E), lambda i: (0, i)),
    ],
    out_shape=[
        jax.ShapeDtypeStruct((E, D), F32),
        jax.ShapeDtypeStruct((8, E), F32),
    ],
)


@functools.partial(
    pl.kernel,
    mesh=_sc_mesh,
    compiler_params=_sc_params,
    out_type=[
        jax.ShapeDtypeStruct((E,), F32),
        jax.ShapeDtypeStruct((NW * N,), F32),
    ],
    scratch_types=[
        pltpu.VMEM((N,), F32),
        pltpu.VMEM((N,), F32),
        pltpu.VMEM((N,), F32),
        pltpu.VMEM((CH3,), F32),
        pltpu.VMEM((CH3,), I32),
        pltpu.VMEM((CH3,), I32),
        pltpu.VMEM((CH3,), F32),
    ],
)
def _k3(u_hbm, pq_hbm, ei_hbm, ex_hbm, den_hbm,
        p_v, q_v, den_v, u_v, s_v, d_v, ex_v):
    wid = lax.axis_index("s") * 2 + lax.axis_index("c")
    pltpu.sync_copy(pq_hbm.at[pl.ds(0, N)], p_v)
    pltpu.sync_copy(pq_hbm.at[pl.ds(N, N)], q_v)

    def _zero(i, carry):
        den_v[pl.ds(i * 16, 16)] = jnp.zeros((16,), F32)
        return carry

    lax.fori_loop(0, N // 16, _zero, 0)

    def _chunk(c, carry):
        base = wid * EPW + c * CH3
        pltpu.sync_copy(u_hbm.at[pl.ds(base, CH3)], u_v)
        pltpu.sync_copy(ei_hbm.at[pl.ds(base, CH3)], s_v)
        pltpu.sync_copy(ei_hbm.at[pl.ds(E + base, CH3)], d_v)

        def _inner(j, icarry):
            off = j * 16
            s = s_v[pl.ds(off, 16)]
            dd = d_v[pl.ds(off, 16)]
            pg = plsc.load_gather(p_v, [s])
            qg = plsc.load_gather(q_v, [dd])
            a = jnp.maximum(u_v[pl.ds(off, 16)] + pg + qg, 0.0)
            ev = jnp.exp(a)
            ex_v[pl.ds(off, 16)] = ev
            plsc.addupdate_scatter(den_v, [dd], ev)
            return icarry

        lax.fori_loop(0, CH3 // 16, _inner, 0)
        pltpu.sync_copy(ex_v, ex_hbm.at[pl.ds(base, CH3)])
        return carry

    lax.fori_loop(0, EPW // CH3, _chunk, 0)
    pltpu.sync_copy(den_v, den_hbm.at[pl.ds(wid * N, N)])


@functools.partial(
    pl.kernel,
    mesh=_sc_mesh,
    compiler_params=_sc_params,
    out_type=jax.ShapeDtypeStruct((2, N, D), F32),
    scratch_types=[
        pltpu.VMEM((EPW,), I32),
        pltpu.VMEM((EPW,), I32),
        pltpu.VMEM((EPW,), F32),
        pltpu.VMEM((CW, D), F32),
        pltpu.VMEM((CW, D), F32),
        pltpu.VMEM_SHARED((N, D), F32),
        pltpu.SemaphoreType.DMA,
        pltpu.SemaphoreType.DMA,
    ],
)
def _k4(ex_hbm, ei_hbm, zh_hbm, out_hbm,
        s1d, d1d, e1d, rows0, rows1, hsh, semf0, semf1):
    cid = lax.axis_index("c")
    sid = lax.axis_index("s")
    wid = sid * 2 + cid

    for i in range(16):
        for g in range(D // 16):
            rows1[i, pl.ds(g * 16, 16)] = jnp.zeros((16,), F32)
    tstart = sid * RPT
    nz = (RPT // 16) + jnp.where(sid == 15, 1, 0)

    def _zfill(t, carry):
        pltpu.sync_copy(rows1.at[pl.ds(0, 16)], hsh.at[pl.ds(tstart + t * 16, 16)])
        return carry

    lax.fori_loop(0, nz, _zfill, 0)
    plsc.subcore_barrier()

    ebase = wid * EPW
    pltpu.sync_copy(ei_hbm.at[pl.ds(ebase, EPW)], s1d)
    pltpu.sync_copy(ei_hbm.at[pl.ds(E + ebase, EPW)], d1d)
    pltpu.sync_copy(ex_hbm.at[pl.ds(ebase, EPW)], e1d)

    bufs = (rows0, rows1)
    sems = (semf0, semf1)

    def _fetch(c, buf, sem):
        pltpu.async_copy(zh_hbm.at[s1d.at[pl.ds(c * CW, CW)]], buf, sem)

    def _waitf(buf, sem):
        pltpu.make_async_copy(zh_hbm.at[s1d.at[pl.ds(0, CW)]], buf, sem).wait()

    def _process(c, buf):
        jb = c * CW

        def _row(r, rcarry):
            a = plsc.load_gather(e1d, [jnp.full((16,), jb + r, I32)])
            for g in range(D // 16):
                sl = pl.ds(g * 16, 16)
                buf[r, sl] = buf[r, sl] * a
            return rcarry

        lax.fori_loop(0, CW, _row, 0, unroll=2)
        pltpu.sync_copy(buf, hsh.at[d1d.at[pl.ds(jb, CW)]], add=True)

    _fetch(0, rows0, semf0)
    _fetch(1, rows1, semf1)

    def _pair(k, carry):
        for b in range(2):
            c = k * 2 + b
            _waitf(bufs[b], sems[b])
            _process(c, bufs[b])
            if b == 0:
                _fetch(c + 2, bufs[b], sems[b])
            else:
                @pl.when(k != (NCH // 2) - 1)
                def _():
                    _fetch(c + 2, bufs[b], sems[b])
        return carry

    lax.fori_loop(0, NCH // 2, _pair, 0)
    _waitf(rows0, semf0)
    _process(NCH - 1, rows0)

    plsc.subcore_barrier()

    def _cpout(t, carry):
        roff = tstart + t * 16
        pltpu.sync_copy(hsh.at[pl.ds(roff, 16)], out_hbm.at[cid, pl.ds(roff, 16)])
        return carry

    lax.fori_loop(0, nz, _cpout, 0)


@functools.partial(
    pl.kernel,
    mesh=_sc_mesh,
    compiler_params=_sc_params,
    out_type=[
        jax.ShapeDtypeStruct((E, D), F32),
        jax.ShapeDtypeStruct((NW, 2, D), F32),
    ],
    scratch_types=[
        pltpu.VMEM((EPW,), I32),
        pltpu.VMEM((EPW,), I32),
        pltpu.VMEM((CW, D), F32),
        pltpu.VMEM((CW, D), F32),
        pltpu.VMEM((CW, D), F32),
        pltpu.VMEM((CW, D), F32),
        pltpu.VMEM((CW, D), F32),
        pltpu.VMEM((CW, D), F32),
        pltpu.VMEM((CW, D), F32),
        pltpu.VMEM((CW, D), F32),
        pltpu.VMEM((D,), F32),
        pltpu.VMEM((D,), F32),
        pltpu.SemaphoreType.DMA,
        pltpu.SemaphoreType.DMA,
        pltpu.SemaphoreType.DMA,
        pltpu.SemaphoreType.DMA,
    ],
)
def _k6a(c_hbm, as_hbm, ad_hbm, ei_hbm, ep_hbm, sums_hbm,
         s1d, d1d, cbuf0, cbuf1, rs0, rs1, rd0, rd1, obuf0, obuf1,
         st1, st2, semf0, semf1, sems0, sems1):
    cid = lax.axis_index("c")
    sid = lax.axis_index("s")
    wid = sid * 2 + cid

    ebase = wid * EPW
    pltpu.sync_copy(ei_hbm.at[pl.ds(ebase, EPW)], s1d)
    pltpu.sync_copy(ei_hbm.at[pl.ds(E + ebase, EPW)], d1d)

    ins = ((cbuf0, rs0, rd0), (cbuf1, rs1, rd1))
    outs = (obuf0, obuf1)
    fsems = (semf0, semf1)
    ssems = (sems0, sems1)
    zero16 = jnp.zeros((16,), F32)

    def _fetch(c, b):
        jb = c * CW
        cb, rsb, rdb = ins[b]
        pltpu.async_copy(c_hbm.at[pl.ds(ebase + jb, CW)], cb, fsems[b])
        pltpu.async_copy(as_hbm.at[s1d.at[pl.ds(jb, CW)]], rsb, fsems[b])
        pltpu.async_copy(ad_hbm.at[d1d.at[pl.ds(jb, CW)]], rdb, fsems[b])

    def _waitf(b):
        cb, rsb, rdb = ins[b]
        pltpu.make_async_copy(c_hbm.at[pl.ds(0, CW)], cb, fsems[b]).wait()
        pltpu.make_async_copy(as_hbm.at[pl.ds(0, CW)], rsb, fsems[b]).wait()
        pltpu.make_async_copy(ad_hbm.at[pl.ds(0, CW)], rdb, fsems[b]).wait()

    def _waits(b):
        pltpu.make_async_copy(outs[b], ep_hbm.at[pl.ds(0, CW)], ssems[b]).wait()

    def _compute(b, carry):
        cb, rsb, rdb = ins[b]
        ob = outs[b]

        def _row(r, rcarry):
            nxt = []
            for g in range(D // 16):
                sl = pl.ds(g * 16, 16)
                v = cb[r, sl] + rsb[r, sl] + rdb[r, sl]
                ob[r, sl] = v
                nxt.append(rcarry[g] + v)
                nxt.append(rcarry[8 + g] + v * v)
            return tuple(nxt[0::2]) + tuple(nxt[1::2])

        return lax.fori_loop(0, CW, _row, carry)

    _fetch(0, 0)
    _fetch(1, 1)
    init = tuple(zero16 for _ in range(16))

    def _pair(k, carry):
        for b in range(2):
            c = k * 2 + b
            _waitf(b)

            @pl.when(k != 0)
            def _():
                _waits(b)

            carry = _compute(b, carry)
            if b == 0:
                _fetch(c + 2, b)
            else:
                @pl.when(k != (NCH // 2) - 1)
                def _():
                    _fetch(c + 2, b)
            pltpu.async_copy(outs[b], ep_hbm.at[pl.ds(ebase + c * CW, CW)],
                             ssems[b])
        return carry

    carry = lax.fori_loop(0, NCH // 2, _pair, init)
    _waitf(0)
    _waits(0)
    carry = _compute(0, carry)
    pltpu.async_copy(outs[0], ep_hbm.at[pl.ds(ebase + (NCH - 1) * CW, CW)],
                     sems0)
    for g in range(D // 16):
        st1[pl.ds(g * 16, 16)] = carry[g]
        st2[pl.ds(g * 16, 16)] = carry[8 + g]
    _waits(1)
    _waits(0)
    pltpu.sync_copy(st1, sums_hbm.at[wid, 0])
    pltpu.sync_copy(st2, sums_hbm.at[wid, 1])


def _k5_body(hacc_ref, den_ref, g_ref, b_ref, out_ref):
    hacc = hacc_ref[0] + hacc_ref[1]
    den = jnp.sum(den_ref[...], axis=0)
    den = jnp.where(den == 0.0, 1.0, den)
    hn = hacc / den[:, None]
    mu = jnp.mean(hn, axis=0, keepdims=True)
    var = jnp.mean(hn * hn, axis=0, keepdims=True) - mu * mu
    scale = g_ref[...] * lax.rsqrt(var + EPS)
    out_ref[...] = jnp.maximum((hn - mu) * scale + b_ref[...], 0.0)


_k5 = pl.pallas_call(
    _k5_body,
    out_shape=jax.ShapeDtypeStruct((N, D), F32),
)


def _k6b_body(ep_ref, sums_ref, g_ref, b_ref, out_ref):
    s = sums_ref[...]
    s1 = jnp.sum(s[:, 0, :], axis=0, keepdims=True)
    s2 = jnp.sum(s[:, 1, :], axis=0, keepdims=True)
    mu = s1 * (1.0 / E)
    var = s2 * (1.0 / E) - mu * mu
    scale = g_ref[...] * lax.rsqrt(var + EPS)
    out_ref[...] = jnp.maximum((ep_ref[...] - mu) * scale + b_ref[...], 0.0)


_k6b = pl.pallas_call(
    _k6b_body,
    grid=(E // BN_E,),
    in_specs=[
        pl.BlockSpec((BN_E, D), lambda i: (i, 0)),
        pl.BlockSpec((NW, 2, D), lambda i: (0, 0, 0)),
        pl.BlockSpec((1, D), lambda i: (0, 0)),
        pl.BlockSpec((1, D), lambda i: (0, 0)),
    ],
    out_specs=pl.BlockSpec((BN_E, D), lambda i: (i, 0)),
    out_shape=jax.ShapeDtypeStruct((E, D), F32),
)


def kernel(h, e, edge_index, W_h, W_e, W_p, b_p, W_a,
           gamma_h, beta_h, gamma_e, beta_e):
    del b_p
    ei = edge_index.reshape(2 * E)
    wpq = jnp.zeros((D, 8), F32)
    wpq = wpq.at[:, 0].set(W_a[D:2 * D, 0]).at[:, 1].set(W_a[2 * D:, 0])
    wue = jnp.zeros((D, 8), F32).at[:, 0].set(W_a[:D, 0])

    z_h, a_s, a_d, pqt, m_e, mu8 = _k1(h, W_h, W_p[D:], wpq, W_e, W_p[:D], wue)
    c, ut = _k2(e, m_e, mu8)

    ex, den_flat = _k3(ut.reshape(8 * E), pqt.reshape(8 * N), ei)

    eproj, sums = _k6a(c, a_s, a_d, ei)
    hacc2 = _k4(ex, ei, z_h)

    h_out = _k5(hacc2, den_flat.reshape(NW, N), gamma_h.reshape(1, D),
                beta_h.reshape(1, D))
    e_out = _k6b(eproj, sums, gamma_e.reshape(1, D), beta_e.reshape(1, D))
    return h_out, e_out
